# baseline (speedup 1.0000x reference)
import functools

import jax
import jax.numpy as jnp
from jax import lax
from jax.experimental import pallas as pl
from jax.experimental.pallas import tpu as pltpu
from jax.experimental.pallas import tpu_sc as plsc

B = 16384
L = 200
D = 32
FF = 128
V = 1000000

NC = 2
NS = 16
NW = NC * NS
BPW = B // NW
LA = 104
LB = L - LA
CH = 128
NCH = BPW // CH

VB4 = 512
NJ = -(-V // (4 * VB4))
NBLK = V // VB4
V4 = NJ * 4 * VB4
TAIL0 = (NJ - 1) * 4 * VB4


def _fmt_body(x0, x1, x2, x3, a0, a1, a2, a3, o_ref):
    last = pl.program_id(0) == NJ - 1
    eye = jnp.eye(D, dtype=jnp.float32)

    def tr(xr):
        return lax.dot_general(xr[...], eye, (((0,), (0,)), ((), ())),
                               preferred_element_type=jnp.float32,
                               precision=lax.Precision.HIGHEST)

    ys = [jnp.where(last, tr(a), tr(x)) for x, a in ((x0, a0), (x1, a1), (x2, a2), (x3, a3))]
    o_ref[...] = jnp.concatenate(ys, axis=1)


def _fmt_tc(embed_t, aux):
    def spec(q):
        return pl.BlockSpec(
            (D, VB4),
            lambda j, q=q: (0, jnp.minimum(4 * j + q, NBLK - 2)),
        )

    def aspec(q):
        return pl.BlockSpec((D, VB4), lambda j, q=q: (0, q))

    return pl.pallas_call(
        _fmt_body,
        grid=(NJ,),
        in_specs=[spec(0), spec(1), spec(2), spec(3),
                  aspec(0), aspec(1), aspec(2), aspec(3)],
        out_specs=pl.BlockSpec((VB4, 4 * D), lambda j: (j, 0)),
        out_shape=jax.ShapeDtypeStruct((NJ * VB4, 4 * D), jnp.float32),
    )(embed_t, embed_t, embed_t, embed_t, aux, aux, aux, aux)


def _pool_sc(tokens, table_lin):
    mesh = plsc.VectorSubcoreMesh(core_axis_name="c", subcore_axis_name="s")

    @functools.partial(
        pl.kernel,
        out_type=jax.ShapeDtypeStruct((B * D,), jnp.float32),
        mesh=mesh,
        compiler_params=pltpu.CompilerParams(use_tc_tiling_on_sc=False),
        scratch_types=[
            pltpu.VMEM((CH, L), jnp.int32),
            pltpu.VMEM((2, L), jnp.int32),
            pltpu.VMEM((2, L, D), jnp.float32),
            pltpu.VMEM((BPW * D,), jnp.float32),
            pltpu.SemaphoreType.DMA,
            pltpu.SemaphoreType.DMA,
        ],
    )
    def pool(tokens_hbm, table_hbm, out_hbm, idx_v, idx2_v, rows_v, pooled_v,
             sem0, sem1):
        wid = lax.axis_index("s") * NC + lax.axis_index("c")
        base = wid * BPW
        sems = (sem0, sem1)

        def remap(s, buf):
            def fix(o_src, o_dst):
                v = idx_v[s, pl.ds(o_src, 16)]
                u = ((v & ~2047) | ((v & 511) << 2)
                     | (lax.shift_right_logical(v, 9) & 3))
                idx2_v[buf, pl.ds(o_dst, 16)] = u

            for i in range(12):
                fix(16 * i, 16 * i)
            fix(L - 16, L - 16)

        def issue(s, buf):
            sem = sems[buf]
            a = pltpu.async_copy(
                table_hbm.at[idx2_v.at[buf, pl.ds(0, LA)]],
                rows_v.at[buf, pl.ds(0, LA)], sem)
            b = pltpu.async_copy(
                table_hbm.at[idx2_v.at[buf, pl.ds(LA, LB)]],
                rows_v.at[buf, pl.ds(LA, LB)], sem)
            return a, b

        def reduce_store(c, s, buf, descs):
            descs[0].wait()
            descs[1].wait()

            def red(r, accs):
                a0, a1 = accs
                a0 = a0 + rows_v[buf, r, pl.ds(0, 16)]
                a1 = a1 + rows_v[buf, r, pl.ds(16, 16)]
                return (a0, a1)

            z = jnp.zeros((16,), jnp.float32)
            a0, a1 = lax.fori_loop(0, L, red, (z, z), unroll=8)
            scale = jnp.float32(1.0 / L)
            o = (s + c * CH) * D
            pooled_v[pl.ds(o, 16)] = a0 * scale
            pooled_v[pl.ds(o + 16, 16)] = a1 * scale

        for c in range(NCH):
            pltpu.sync_copy(tokens_hbm.at[pl.ds(base + c * CH, CH)], idx_v)
            remap(0, 0)
            d0 = issue(0, 0)

            @pl.loop(0, CH, step=2)
            def _pair(k):
                remap(k + 1, 1)
                da = issue(k + 1, 1)
                reduce_store(c, k, 0, d0)

                @pl.when(k < CH - 2)
                def _():
                    remap(k + 2, 0)
                    issue(k + 2, 0)

                reduce_store(c, k + 1, 1, da)

        pltpu.sync_copy(pooled_v, out_hbm.at[pl.ds(base * D, BPW * D)])

    return pool(tokens, table_lin)


BM = 2048


def _mlp_body(x_ref, w1_ref, b1_ref, w2_ref, b2_ref, o_ref):
    x = x_ref[...]
    h = jnp.maximum(
        jnp.dot(x, w1_ref[...], preferred_element_type=jnp.float32) + b1_ref[...],
        0.0,
    )
    o_ref[...] = jnp.dot(h, w2_ref[...], preferred_element_type=jnp.float32) + b2_ref[...]


def _mlp_tc(x, W1, b1, W2, b2):
    return pl.pallas_call(
        _mlp_body,
        grid=(B // BM,),
        in_specs=[
            pl.BlockSpec((BM, D), lambda i: (i, 0)),
            pl.BlockSpec((D, FF), lambda i: (0, 0)),
            pl.BlockSpec((1, FF), lambda i: (0, 0)),
            pl.BlockSpec((FF, 1), lambda i: (0, 0)),
            pl.BlockSpec((1, 1), lambda i: (0, 0)),
        ],
        out_specs=pl.BlockSpec((BM, 1), lambda i: (i, 0)),
        out_shape=jax.ShapeDtypeStruct((B, 1), jnp.float32),
    )(x, W1, b1.reshape(1, FF), W2, b2.reshape(1, 1))


def kernel(tokens, embed, W1, b1, W2, b2):
    embed_t = embed.T
    aux = jnp.pad(embed_t[:, TAIL0:], ((0, 0), (0, 4 * VB4 - (V - TAIL0))))
    table_lin = _fmt_tc(embed_t, aux).reshape(V4, D)
    pooled = _pool_sc(tokens.astype(jnp.int32), table_lin).reshape(B, D)
    out = _mlp_tc(pooled, W1, b1, W2, b2)
    return out[:, 0]


# speedup vs baseline: 1.6211x; 1.6211x over previous
import functools

import jax
import jax.numpy as jnp
from jax import lax
from jax.experimental import pallas as pl
from jax.experimental.pallas import tpu as pltpu
from jax.experimental.pallas import tpu_sc as plsc

B = 16384
L = 200
D = 32
FF = 128
V = 1000000

NC = 2
NS = 16
NW = NC * NS
BPW = B // NW
LA = 104
LB = L - LA
CH = 128
NCH = BPW // CH

VB4 = 512
NJ = -(-V // (4 * VB4))
NBLK = V // VB4
V4 = NJ * 4 * VB4
TAIL0 = (NJ - 1) * 4 * VB4


def _fmt_body(x0, x1, x2, x3, a0, a1, a2, a3, o_ref):
    last = pl.program_id(0) == NJ - 1
    eye = jnp.eye(D, dtype=jnp.float32)

    def tr(xr):
        return lax.dot_general(xr[...], eye, (((0,), (0,)), ((), ())),
                               preferred_element_type=jnp.float32)

    ys = [jnp.where(last, tr(a), tr(x)) for x, a in ((x0, a0), (x1, a1), (x2, a2), (x3, a3))]
    o_ref[...] = jnp.concatenate(ys, axis=1)


def _fmt_tc(embed_t, aux):
    def spec(q):
        return pl.BlockSpec(
            (D, VB4),
            lambda j, q=q: (0, jnp.minimum(4 * j + q, NBLK - 2)),
        )

    def aspec(q):
        return pl.BlockSpec((D, VB4), lambda j, q=q: (0, q))

    return pl.pallas_call(
        _fmt_body,
        grid=(NJ,),
        in_specs=[spec(0), spec(1), spec(2), spec(3),
                  aspec(0), aspec(1), aspec(2), aspec(3)],
        out_specs=pl.BlockSpec((VB4, 4 * D), lambda j: (j, 0)),
        out_shape=jax.ShapeDtypeStruct((NJ * VB4, 4 * D), jnp.float32),
    )(embed_t, embed_t, embed_t, embed_t, aux, aux, aux, aux)


def _pool_sc(tokens, table_lin):
    mesh = plsc.VectorSubcoreMesh(core_axis_name="c", subcore_axis_name="s")

    @functools.partial(
        pl.kernel,
        out_type=jax.ShapeDtypeStruct((B * D,), jnp.float32),
        mesh=mesh,
        compiler_params=pltpu.CompilerParams(use_tc_tiling_on_sc=False),
        scratch_types=[
            pltpu.VMEM((CH, L), jnp.int32),
            pltpu.VMEM((2, L), jnp.int32),
            pltpu.VMEM((2, L, D), jnp.float32),
            pltpu.VMEM((BPW * D,), jnp.float32),
            pltpu.SemaphoreType.DMA,
            pltpu.SemaphoreType.DMA,
        ],
    )
    def pool(tokens_hbm, table_hbm, out_hbm, idx_v, idx2_v, rows_v, pooled_v,
             sem0, sem1):
        wid = lax.axis_index("s") * NC + lax.axis_index("c")
        base = wid * BPW
        sems = (sem0, sem1)

        def remap(s, buf):
            def fix(o_src, o_dst):
                v = idx_v[s, pl.ds(o_src, 16)]
                u = ((v & ~2047) | ((v & 511) << 2)
                     | (lax.shift_right_logical(v, 9) & 3))
                idx2_v[buf, pl.ds(o_dst, 16)] = u

            for i in range(12):
                fix(16 * i, 16 * i)
            fix(L - 16, L - 16)

        def issue(s, buf):
            sem = sems[buf]
            a = pltpu.async_copy(
                table_hbm.at[idx2_v.at[buf, pl.ds(0, LA)]],
                rows_v.at[buf, pl.ds(0, LA)], sem)
            b = pltpu.async_copy(
                table_hbm.at[idx2_v.at[buf, pl.ds(LA, LB)]],
                rows_v.at[buf, pl.ds(LA, LB)], sem)
            return a, b

        def reduce_store(c, s, buf, descs):
            descs[0].wait()
            descs[1].wait()

            def red(r, accs):
                a0, a1 = accs
                a0 = a0 + rows_v[buf, r, pl.ds(0, 16)]
                a1 = a1 + rows_v[buf, r, pl.ds(16, 16)]
                return (a0, a1)

            z = jnp.zeros((16,), jnp.float32)
            a0, a1 = lax.fori_loop(0, L, red, (z, z), unroll=8)
            scale = jnp.float32(1.0 / L)
            o = (s + c * CH) * D
            pooled_v[pl.ds(o, 16)] = a0 * scale
            pooled_v[pl.ds(o + 16, 16)] = a1 * scale

        for c in range(NCH):
            pltpu.sync_copy(tokens_hbm.at[pl.ds(base + c * CH, CH)], idx_v)
            remap(0, 0)
            d0 = issue(0, 0)

            @pl.loop(0, CH, step=2)
            def _pair(k):
                remap(k + 1, 1)
                da = issue(k + 1, 1)
                reduce_store(c, k, 0, d0)

                @pl.when(k < CH - 2)
                def _():
                    remap(k + 2, 0)
                    issue(k + 2, 0)

                reduce_store(c, k + 1, 1, da)

        pltpu.sync_copy(pooled_v, out_hbm.at[pl.ds(base * D, BPW * D)])

    return pool(tokens, table_lin)


BM = 2048


def _mlp_body(x_ref, w1_ref, b1_ref, w2_ref, b2_ref, o_ref):
    x = x_ref[...]
    h = jnp.maximum(
        jnp.dot(x, w1_ref[...], preferred_element_type=jnp.float32) + b1_ref[...],
        0.0,
    )
    o_ref[...] = jnp.dot(h, w2_ref[...], preferred_element_type=jnp.float32) + b2_ref[...]


def _mlp_tc(x, W1, b1, W2, b2):
    return pl.pallas_call(
        _mlp_body,
        grid=(B // BM,),
        in_specs=[
            pl.BlockSpec((BM, D), lambda i: (i, 0)),
            pl.BlockSpec((D, FF), lambda i: (0, 0)),
            pl.BlockSpec((1, FF), lambda i: (0, 0)),
            pl.BlockSpec((FF, 1), lambda i: (0, 0)),
            pl.BlockSpec((1, 1), lambda i: (0, 0)),
        ],
        out_specs=pl.BlockSpec((BM, 1), lambda i: (i, 0)),
        out_shape=jax.ShapeDtypeStruct((B, 1), jnp.float32),
    )(x, W1, b1.reshape(1, FF), W2, b2.reshape(1, 1))


def kernel(tokens, embed, W1, b1, W2, b2):
    embed_t = embed.T
    aux = jnp.pad(embed_t[:, TAIL0:], ((0, 0), (0, 4 * VB4 - (V - TAIL0))))
    table_lin = _fmt_tc(embed_t, aux).reshape(V4, D)
    pooled = _pool_sc(tokens.astype(jnp.int32), table_lin).reshape(B, D)
    out = _mlp_tc(pooled, W1, b1, W2, b2)
    return out[:, 0]


# speedup vs baseline: 1.8197x; 1.1225x over previous
import functools

import jax
import jax.numpy as jnp
from jax import lax
from jax.experimental import pallas as pl
from jax.experimental.pallas import tpu as pltpu
from jax.experimental.pallas import tpu_sc as plsc

B = 16384
L = 200
D = 32
FF = 128
V = 1000000

NC = 2
NS = 16
NW = NC * NS
BPW = B // NW
LA = 104
LB = L - LA
CH = 128
NCH = BPW // CH

VB4 = 2048
LGVB = 11
NJ = -(-V // (4 * VB4))
NBLK = V // VB4
V4 = NJ * 4 * VB4
TAIL0 = (NJ - 1) * 4 * VB4


def _fmt_body(x0, x1, x2, x3, a0, a1, a2, a3, o_ref):
    last = pl.program_id(0) == NJ - 1
    eye = jnp.eye(D, dtype=jnp.float32)

    def tr(xr):
        return lax.dot_general(xr[...], eye, (((0,), (0,)), ((), ())),
                               preferred_element_type=jnp.float32)

    ys = [jnp.where(last, tr(a), tr(x)) for x, a in ((x0, a0), (x1, a1), (x2, a2), (x3, a3))]
    o_ref[...] = jnp.concatenate(ys, axis=1)


def _fmt_tc(embed_t, aux):
    def spec(q):
        return pl.BlockSpec(
            (D, VB4),
            lambda j, q=q: (0, jnp.minimum(4 * j + q, NBLK - 2)),
        )

    def aspec(q):
        return pl.BlockSpec((D, VB4), lambda j, q=q: (0, q))

    return pl.pallas_call(
        _fmt_body,
        grid=(NJ,),
        in_specs=[spec(0), spec(1), spec(2), spec(3),
                  aspec(0), aspec(1), aspec(2), aspec(3)],
        out_specs=pl.BlockSpec((VB4, 4 * D), lambda j: (j, 0)),
        out_shape=jax.ShapeDtypeStruct((NJ * VB4, 4 * D), jnp.float32),
    )(embed_t, embed_t, embed_t, embed_t, aux, aux, aux, aux)


def _pool_sc(tokens, table_lin):
    mesh = plsc.VectorSubcoreMesh(core_axis_name="c", subcore_axis_name="s")

    @functools.partial(
        pl.kernel,
        out_type=jax.ShapeDtypeStruct((B * D,), jnp.float32),
        mesh=mesh,
        compiler_params=pltpu.CompilerParams(use_tc_tiling_on_sc=False),
        scratch_types=[
            pltpu.VMEM((CH, L), jnp.int32),
            pltpu.VMEM((2, L), jnp.int32),
            pltpu.VMEM((2, L, D), jnp.float32),
            pltpu.VMEM((BPW * D,), jnp.float32),
            pltpu.SemaphoreType.DMA,
            pltpu.SemaphoreType.DMA,
        ],
    )
    def pool(tokens_hbm, table_hbm, out_hbm, idx_v, idx2_v, rows_v, pooled_v,
             sem0, sem1):
        wid = lax.axis_index("s") * NC + lax.axis_index("c")
        base = wid * BPW
        sems = (sem0, sem1)

        def remap(s, buf):
            def fix(o_src, o_dst):
                v = idx_v[s, pl.ds(o_src, 16)]
                u = ((v & ~(4 * VB4 - 1)) | ((v & (VB4 - 1)) << 2)
                     | (lax.shift_right_logical(v, LGVB) & 3))
                idx2_v[buf, pl.ds(o_dst, 16)] = u

            for i in range(12):
                fix(16 * i, 16 * i)
            fix(L - 16, L - 16)

        def issue(s, buf):
            sem = sems[buf]
            a = pltpu.async_copy(
                table_hbm.at[idx2_v.at[buf, pl.ds(0, LA)]],
                rows_v.at[buf, pl.ds(0, LA)], sem)
            b = pltpu.async_copy(
                table_hbm.at[idx2_v.at[buf, pl.ds(LA, LB)]],
                rows_v.at[buf, pl.ds(LA, LB)], sem)
            return a, b

        def reduce_store(c, s, buf, descs):
            descs[0].wait()
            descs[1].wait()

            def red(r, accs):
                a0, a1 = accs
                a0 = a0 + rows_v[buf, r, pl.ds(0, 16)]
                a1 = a1 + rows_v[buf, r, pl.ds(16, 16)]
                return (a0, a1)

            z = jnp.zeros((16,), jnp.float32)
            a0, a1 = lax.fori_loop(0, L, red, (z, z), unroll=8)
            scale = jnp.float32(1.0 / L)
            o = (s + c * CH) * D
            pooled_v[pl.ds(o, 16)] = a0 * scale
            pooled_v[pl.ds(o + 16, 16)] = a1 * scale

        for c in range(NCH):
            pltpu.sync_copy(tokens_hbm.at[pl.ds(base + c * CH, CH)], idx_v)
            remap(0, 0)
            d0 = issue(0, 0)

            @pl.loop(0, CH, step=2)
            def _pair(k):
                remap(k + 1, 1)
                da = issue(k + 1, 1)
                reduce_store(c, k, 0, d0)

                @pl.when(k < CH - 2)
                def _():
                    remap(k + 2, 0)
                    issue(k + 2, 0)

                reduce_store(c, k + 1, 1, da)

        pltpu.sync_copy(pooled_v, out_hbm.at[pl.ds(base * D, BPW * D)])

    return pool(tokens, table_lin)


BM = 2048


def _mlp_body(x_ref, w1_ref, b1_ref, w2_ref, b2_ref, o_ref):
    x = x_ref[...]
    h = jnp.maximum(
        jnp.dot(x, w1_ref[...], preferred_element_type=jnp.float32) + b1_ref[...],
        0.0,
    )
    o_ref[...] = jnp.dot(h, w2_ref[...], preferred_element_type=jnp.float32) + b2_ref[...]


def _mlp_tc(x, W1, b1, W2, b2):
    return pl.pallas_call(
        _mlp_body,
        grid=(B // BM,),
        in_specs=[
            pl.BlockSpec((BM, D), lambda i: (i, 0)),
            pl.BlockSpec((D, FF), lambda i: (0, 0)),
            pl.BlockSpec((1, FF), lambda i: (0, 0)),
            pl.BlockSpec((FF, 1), lambda i: (0, 0)),
            pl.BlockSpec((1, 1), lambda i: (0, 0)),
        ],
        out_specs=pl.BlockSpec((BM, 1), lambda i: (i, 0)),
        out_shape=jax.ShapeDtypeStruct((B, 1), jnp.float32),
    )(x, W1, b1.reshape(1, FF), W2, b2.reshape(1, 1))


def kernel(tokens, embed, W1, b1, W2, b2):
    embed_t = embed.T
    aux = jnp.pad(embed_t[:, TAIL0:], ((0, 0), (0, 4 * VB4 - (V - TAIL0))))
    table_lin = _fmt_tc(embed_t, aux).reshape(V4, D)
    pooled = _pool_sc(tokens.astype(jnp.int32), table_lin).reshape(B, D)
    out = _mlp_tc(pooled, W1, b1, W2, b2)
    return out[:, 0]


# speedup vs baseline: 1.8233x; 1.0020x over previous
import functools

import jax
import jax.numpy as jnp
from jax import lax
from jax.experimental import pallas as pl
from jax.experimental.pallas import tpu as pltpu
from jax.experimental.pallas import tpu_sc as plsc

B = 16384
L = 200
D = 32
FF = 128
V = 1000000

NC = 2
NS = 16
NW = NC * NS
BPW = B // NW
LA = 104
LB = L - LA
CH = 128
NCH = BPW // CH

VB4 = 2048
LGVB = 11
NJ = -(-V // (4 * VB4))
NBLK = V // VB4
V4 = NJ * 4 * VB4
TAIL0 = (NJ - 1) * 4 * VB4


def _fmt_body(x0, x1, x2, x3, a0, a1, a2, a3, o_ref):
    last = pl.program_id(0) == NJ - 1
    eye = jnp.eye(D, dtype=jnp.float32)

    def tr(xr):
        return lax.dot_general(xr[...], eye, (((0,), (0,)), ((), ())),
                               preferred_element_type=jnp.float32)

    ys = [jnp.where(last, tr(a), tr(x)) for x, a in ((x0, a0), (x1, a1), (x2, a2), (x3, a3))]
    o_ref[...] = jnp.concatenate(ys, axis=1)


def _fmt_tc(embed_t, aux):
    def spec(q):
        return pl.BlockSpec(
            (D, VB4),
            lambda j, q=q: (0, jnp.minimum(4 * j + q, NBLK - 1)),
        )

    def aspec(q):
        return pl.BlockSpec((D, VB4), lambda j, q=q: (0, q))

    return pl.pallas_call(
        _fmt_body,
        grid=(NJ,),
        in_specs=[spec(0), spec(1), spec(2), spec(3),
                  aspec(0), aspec(1), aspec(2), aspec(3)],
        out_specs=pl.BlockSpec((VB4, 4 * D), lambda j: (j, 0)),
        out_shape=jax.ShapeDtypeStruct((NJ * VB4, 4 * D), jnp.float32),
    )(embed_t, embed_t, embed_t, embed_t, aux, aux, aux, aux)


def _pool_sc(tokens, table_lin):
    mesh = plsc.VectorSubcoreMesh(core_axis_name="c", subcore_axis_name="s")

    @functools.partial(
        pl.kernel,
        out_type=jax.ShapeDtypeStruct((B * D,), jnp.float32),
        mesh=mesh,
        compiler_params=pltpu.CompilerParams(use_tc_tiling_on_sc=False),
        scratch_types=[
            pltpu.VMEM((CH, L), jnp.int32),
            pltpu.VMEM((2, L), jnp.int32),
            pltpu.VMEM((2, L, D), jnp.float32),
            pltpu.VMEM((BPW * D,), jnp.float32),
            pltpu.SemaphoreType.DMA,
            pltpu.SemaphoreType.DMA,
        ],
    )
    def pool(tokens_hbm, table_hbm, out_hbm, idx_v, idx2_v, rows_v, pooled_v,
             sem0, sem1):
        wid = lax.axis_index("s") * NC + lax.axis_index("c")
        base = wid * BPW
        sems = (sem0, sem1)

        def remap(s, buf):
            def fix(o_src, o_dst):
                v = idx_v[s, pl.ds(o_src, 16)]
                u = ((v & ~(4 * VB4 - 1)) | ((v & (VB4 - 1)) << 2)
                     | (lax.shift_right_logical(v, LGVB) & 3))
                idx2_v[buf, pl.ds(o_dst, 16)] = u

            for i in range(12):
                fix(16 * i, 16 * i)
            fix(L - 16, L - 16)

        def issue(s, buf):
            sem = sems[buf]
            a = pltpu.async_copy(
                table_hbm.at[idx2_v.at[buf, pl.ds(0, LA)]],
                rows_v.at[buf, pl.ds(0, LA)], sem)
            b = pltpu.async_copy(
                table_hbm.at[idx2_v.at[buf, pl.ds(LA, LB)]],
                rows_v.at[buf, pl.ds(LA, LB)], sem)
            return a, b

        def reduce_store(c, s, buf, descs):
            descs[0].wait()
            descs[1].wait()

            def red(r, accs):
                a0, a1 = accs
                a0 = a0 + rows_v[buf, r, pl.ds(0, 16)]
                a1 = a1 + rows_v[buf, r, pl.ds(16, 16)]
                return (a0, a1)

            z = jnp.zeros((16,), jnp.float32)
            a0, a1 = lax.fori_loop(0, L, red, (z, z), unroll=8)
            scale = jnp.float32(1.0 / L)
            o = (s + c * CH) * D
            pooled_v[pl.ds(o, 16)] = a0 * scale
            pooled_v[pl.ds(o + 16, 16)] = a1 * scale

        for c in range(NCH):
            pltpu.sync_copy(tokens_hbm.at[pl.ds(base + c * CH, CH)], idx_v)
            remap(0, 0)
            d0 = issue(0, 0)

            @pl.loop(0, CH, step=2)
            def _pair(k):
                remap(k + 1, 1)
                da = issue(k + 1, 1)
                reduce_store(c, k, 0, d0)

                @pl.when(k < CH - 2)
                def _():
                    remap(k + 2, 0)
                    issue(k + 2, 0)

                reduce_store(c, k + 1, 1, da)

        pltpu.sync_copy(pooled_v, out_hbm.at[pl.ds(base * D, BPW * D)])

    return pool(tokens, table_lin)


BM = 2048


def _mlp_body(x_ref, w1_ref, b1_ref, w2_ref, b2_ref, o_ref):
    x = x_ref[...]
    h = jnp.maximum(
        jnp.dot(x, w1_ref[...], preferred_element_type=jnp.float32) + b1_ref[...],
        0.0,
    )
    o_ref[...] = jnp.dot(h, w2_ref[...], preferred_element_type=jnp.float32) + b2_ref[...]


def _mlp_tc(x, W1, b1, W2, b2):
    return pl.pallas_call(
        _mlp_body,
        grid=(B // BM,),
        in_specs=[
            pl.BlockSpec((BM, D), lambda i: (i, 0)),
            pl.BlockSpec((D, FF), lambda i: (0, 0)),
            pl.BlockSpec((1, FF), lambda i: (0, 0)),
            pl.BlockSpec((FF, 1), lambda i: (0, 0)),
            pl.BlockSpec((1, 1), lambda i: (0, 0)),
        ],
        out_specs=pl.BlockSpec((BM, 1), lambda i: (i, 0)),
        out_shape=jax.ShapeDtypeStruct((B, 1), jnp.float32),
    )(x, W1, b1.reshape(1, FF), W2, b2.reshape(1, 1))


def kernel(tokens, embed, W1, b1, W2, b2):
    embed_t = embed.T
    aux = jnp.pad(embed_t[:, TAIL0:], ((0, 0), (0, 4 * VB4 - (V - TAIL0))))
    table_lin = _fmt_tc(embed_t, aux).reshape(V4, D)
    pooled = _pool_sc(tokens.astype(jnp.int32), table_lin).reshape(B, D)
    out = _mlp_tc(pooled, W1, b1, W2, b2)
    return out[:, 0]


# speedup vs baseline: 1.8965x; 1.0401x over previous
import functools

import jax
import jax.numpy as jnp
from jax import lax
from jax.experimental import pallas as pl
from jax.experimental.pallas import tpu as pltpu
from jax.experimental.pallas import tpu_sc as plsc

B = 16384
L = 200
D = 32
FF = 128
V = 1000000

NC = 2
NS = 16
NW = NC * NS
BPW = B // NW
LA = 104
LB = L - LA
CH = 128
NCH = BPW // CH
KACC = 40

VB4 = 2048
LGVB = 11
NJ = -(-V // (4 * VB4))
NBLK = V // VB4
V4 = NJ * 4 * VB4
TAIL0 = (NJ - 1) * 4 * VB4


def _fmt_body(x0, x1, x2, x3, a0, a1, a2, a3, o_ref):
    last = pl.program_id(0) == NJ - 1
    eye = jnp.eye(D, dtype=jnp.float32)

    def tr(xr):
        return lax.dot_general(xr[...], eye, (((0,), (0,)), ((), ())),
                               preferred_element_type=jnp.float32)

    ys = [jnp.where(last, tr(a), tr(x)) for x, a in ((x0, a0), (x1, a1), (x2, a2), (x3, a3))]
    o_ref[...] = jnp.concatenate(ys, axis=1)


def _fmt_tc(embed_t, aux):
    def spec(q):
        return pl.BlockSpec(
            (D, VB4),
            lambda j, q=q: (0, jnp.minimum(4 * j + q, NBLK - 1)),
        )

    def aspec(q):
        return pl.BlockSpec((D, VB4), lambda j, q=q: (0, q))

    return pl.pallas_call(
        _fmt_body,
        grid=(NJ,),
        in_specs=[spec(0), spec(1), spec(2), spec(3),
                  aspec(0), aspec(1), aspec(2), aspec(3)],
        out_specs=pl.BlockSpec((VB4, 4 * D), lambda j: (j, 0)),
        out_shape=jax.ShapeDtypeStruct((NJ * VB4, 4 * D), jnp.float32),
    )(embed_t, embed_t, embed_t, embed_t, aux, aux, aux, aux)


def _pool_sc(tokens, table_lin):
    mesh = plsc.VectorSubcoreMesh(core_axis_name="c", subcore_axis_name="s")

    @functools.partial(
        pl.kernel,
        out_type=jax.ShapeDtypeStruct((B * D,), jnp.float32),
        mesh=mesh,
        compiler_params=pltpu.CompilerParams(use_tc_tiling_on_sc=False),
        scratch_types=[
            pltpu.VMEM((CH, L), jnp.int32),
            pltpu.VMEM((2, L), jnp.int32),
            pltpu.VMEM((2, KACC, D), jnp.float32),
            pltpu.VMEM((BPW * D,), jnp.float32),
            pltpu.SemaphoreType.DMA,
            pltpu.SemaphoreType.DMA,
        ],
    )
    def pool(tokens_hbm, table_hbm, out_hbm, idx_v, idx2_v, rows_v, pooled_v,
             sem0, sem1):
        wid = lax.axis_index("s") * NC + lax.axis_index("c")
        base = wid * BPW
        sems = (sem0, sem1)
        zvec = jnp.zeros((16,), jnp.float32)

        def zero(buf):
            for i in range(KACC):
                rows_v[buf, i, pl.ds(0, 16)] = zvec
                rows_v[buf, i, pl.ds(16, 16)] = zvec

        def remap(s, buf):
            def fix(o_src, o_dst):
                v = idx_v[s, pl.ds(o_src, 16)]
                u = ((v & ~(4 * VB4 - 1)) | ((v & (VB4 - 1)) << 2)
                     | (lax.shift_right_logical(v, LGVB) & 3))
                idx2_v[buf, pl.ds(o_dst, 16)] = u

            for i in range(12):
                fix(16 * i, 16 * i)
            fix(L - 16, L - 16)

        def issue(s, buf):
            sem = sems[buf]
            return tuple(
                pltpu.async_copy(
                    table_hbm.at[idx2_v.at[buf, pl.ds(g * KACC, KACC)]],
                    rows_v.at[buf], sem, add=True)
                for g in range(L // KACC))

        def reduce_store(c, s, buf, descs):
            for d in descs:
                d.wait()

            def red(r, accs):
                a0, a1 = accs
                a0 = a0 + rows_v[buf, r, pl.ds(0, 16)]
                a1 = a1 + rows_v[buf, r, pl.ds(16, 16)]
                return (a0, a1)

            a0, a1 = lax.fori_loop(0, KACC, red, (zvec, zvec), unroll=8)
            zero(buf)
            scale = jnp.float32(1.0 / L)
            o = (s + c * CH) * D
            pooled_v[pl.ds(o, 16)] = a0 * scale
            pooled_v[pl.ds(o + 16, 16)] = a1 * scale

        zero(0)
        zero(1)
        for c in range(NCH):
            pltpu.sync_copy(tokens_hbm.at[pl.ds(base + c * CH, CH)], idx_v)
            remap(0, 0)
            d0 = issue(0, 0)

            @pl.loop(0, CH, step=2)
            def _pair(k):
                remap(k + 1, 1)
                da = issue(k + 1, 1)
                reduce_store(c, k, 0, d0)

                @pl.when(k < CH - 2)
                def _():
                    remap(k + 2, 0)
                    issue(k + 2, 0)

                reduce_store(c, k + 1, 1, da)

        pltpu.sync_copy(pooled_v, out_hbm.at[pl.ds(base * D, BPW * D)])

    return pool(tokens, table_lin)


BM = 2048


def _mlp_body(x_ref, w1_ref, b1_ref, w2_ref, b2_ref, o_ref):
    x = x_ref[...]
    h = jnp.maximum(
        jnp.dot(x, w1_ref[...], preferred_element_type=jnp.float32) + b1_ref[...],
        0.0,
    )
    o_ref[...] = jnp.dot(h, w2_ref[...], preferred_element_type=jnp.float32) + b2_ref[...]


def _mlp_tc(x, W1, b1, W2, b2):
    return pl.pallas_call(
        _mlp_body,
        grid=(B // BM,),
        in_specs=[
            pl.BlockSpec((BM, D), lambda i: (i, 0)),
            pl.BlockSpec((D, FF), lambda i: (0, 0)),
            pl.BlockSpec((1, FF), lambda i: (0, 0)),
            pl.BlockSpec((FF, 1), lambda i: (0, 0)),
            pl.BlockSpec((1, 1), lambda i: (0, 0)),
        ],
        out_specs=pl.BlockSpec((BM, 1), lambda i: (i, 0)),
        out_shape=jax.ShapeDtypeStruct((B, 1), jnp.float32),
    )(x, W1, b1.reshape(1, FF), W2, b2.reshape(1, 1))


def kernel(tokens, embed, W1, b1, W2, b2):
    embed_t = embed.T
    aux = jnp.pad(embed_t[:, TAIL0:], ((0, 0), (0, 4 * VB4 - (V - TAIL0))))
    table_lin = _fmt_tc(embed_t, aux).reshape(V4, D)
    pooled = _pool_sc(tokens.astype(jnp.int32), table_lin).reshape(B, D)
    out = _mlp_tc(pooled, W1, b1, W2, b2)
    return out[:, 0]


# speedup vs baseline: 2.8340x; 1.4944x over previous
import functools

import jax
import jax.numpy as jnp
from jax import lax
from jax.experimental import pallas as pl
from jax.experimental.pallas import tpu as pltpu
from jax.experimental.pallas import tpu_sc as plsc

B = 16384
L = 200
D = 32
FF = 128
V = 1000000

NC = 2
NS = 16
NW = NC * NS
BPW = B // NW
LA = 104
LB = L - LA
CH = 128
NCH = BPW // CH
KACC = 40

VB4 = 2048
LGVB = 11
NJ = -(-V // (4 * VB4))
NBLK = V // VB4
V4 = NJ * 4 * VB4
TAIL0 = (NJ - 1) * 4 * VB4


def _fmt_body(x0, x1, x2, x3, a0, a1, a2, a3, o_ref):
    last = pl.program_id(0) == NJ - 1
    xs = [jnp.where(last, a[...], x[...])
          for x, a in ((x0, a0), (x1, a1), (x2, a2), (x3, a3))]
    xcat = jnp.concatenate(xs, axis=0)
    eye = jnp.eye(4 * D, dtype=jnp.float32)
    o_ref[...] = lax.dot_general(xcat, eye, (((0,), (0,)), ((), ())),
                                 preferred_element_type=jnp.float32)


def _fmt_tc(embed_t, aux):
    def spec(q):
        return pl.BlockSpec(
            (D, VB4),
            lambda j, q=q: (0, jnp.minimum(4 * j + q, NBLK - 1)),
        )

    def aspec(q):
        return pl.BlockSpec((D, VB4), lambda j, q=q: (0, q))

    return pl.pallas_call(
        _fmt_body,
        grid=(NJ,),
        in_specs=[spec(0), spec(1), spec(2), spec(3),
                  aspec(0), aspec(1), aspec(2), aspec(3)],
        out_specs=pl.BlockSpec((VB4, 4 * D), lambda j: (j, 0)),
        out_shape=jax.ShapeDtypeStruct((NJ * VB4, 4 * D), jnp.float32),
    )(embed_t, embed_t, embed_t, embed_t, aux, aux, aux, aux)


def _pool_sc(tokens, table_lin):
    mesh = plsc.VectorSubcoreMesh(core_axis_name="c", subcore_axis_name="s")

    @functools.partial(
        pl.kernel,
        out_type=jax.ShapeDtypeStruct((B * D,), jnp.float32),
        mesh=mesh,
        compiler_params=pltpu.CompilerParams(use_tc_tiling_on_sc=False),
        scratch_types=[
            pltpu.VMEM((CH, L), jnp.int32),
            pltpu.VMEM((2, L), jnp.int32),
            pltpu.VMEM((2, KACC, D), jnp.float32),
            pltpu.VMEM((BPW * D,), jnp.float32),
            pltpu.SemaphoreType.DMA,
            pltpu.SemaphoreType.DMA,
        ],
    )
    def pool(tokens_hbm, table_hbm, out_hbm, idx_v, idx2_v, rows_v, pooled_v,
             sem0, sem1):
        wid = lax.axis_index("s") * NC + lax.axis_index("c")
        base = wid * BPW
        sems = (sem0, sem1)
        zvec = jnp.zeros((16,), jnp.float32)

        def zero(buf):
            for i in range(KACC):
                rows_v[buf, i, pl.ds(0, 16)] = zvec
                rows_v[buf, i, pl.ds(16, 16)] = zvec

        def remap(s, buf):
            def fix(o_src, o_dst):
                v = idx_v[s, pl.ds(o_src, 16)]
                u = ((v & ~(4 * VB4 - 1)) | ((v & (VB4 - 1)) << 2)
                     | (lax.shift_right_logical(v, LGVB) & 3))
                idx2_v[buf, pl.ds(o_dst, 16)] = u

            for i in range(12):
                fix(16 * i, 16 * i)
            fix(L - 16, L - 16)

        def issue(s, buf):
            sem = sems[buf]
            return tuple(
                pltpu.async_copy(
                    table_hbm.at[idx2_v.at[buf, pl.ds(g * KACC, KACC)]],
                    rows_v.at[buf], sem, add=True)
                for g in range(L // KACC))

        def reduce_store(c, s, buf, descs):
            for d in descs:
                d.wait()

            def red(r, accs):
                a0, a1 = accs
                a0 = a0 + rows_v[buf, r, pl.ds(0, 16)]
                a1 = a1 + rows_v[buf, r, pl.ds(16, 16)]
                return (a0, a1)

            a0, a1 = lax.fori_loop(0, KACC, red, (zvec, zvec), unroll=8)
            zero(buf)
            scale = jnp.float32(1.0 / L)
            o = (s + c * CH) * D
            pooled_v[pl.ds(o, 16)] = a0 * scale
            pooled_v[pl.ds(o + 16, 16)] = a1 * scale

        zero(0)
        zero(1)
        for c in range(NCH):
            pltpu.sync_copy(tokens_hbm.at[pl.ds(base + c * CH, CH)], idx_v)
            remap(0, 0)
            d0 = issue(0, 0)

            @pl.loop(0, CH, step=2)
            def _pair(k):
                remap(k + 1, 1)
                da = issue(k + 1, 1)
                reduce_store(c, k, 0, d0)

                @pl.when(k < CH - 2)
                def _():
                    remap(k + 2, 0)
                    issue(k + 2, 0)

                reduce_store(c, k + 1, 1, da)

        pltpu.sync_copy(pooled_v, out_hbm.at[pl.ds(base * D, BPW * D)])

    return pool(tokens, table_lin)


BM = 2048


def _mlp_body(x_ref, w1_ref, b1_ref, w2_ref, b2_ref, o_ref):
    x = x_ref[...]
    h = jnp.maximum(
        jnp.dot(x, w1_ref[...], preferred_element_type=jnp.float32) + b1_ref[...],
        0.0,
    )
    o_ref[...] = jnp.dot(h, w2_ref[...], preferred_element_type=jnp.float32) + b2_ref[...]


def _mlp_tc(x, W1, b1, W2, b2):
    return pl.pallas_call(
        _mlp_body,
        grid=(B // BM,),
        in_specs=[
            pl.BlockSpec((BM, D), lambda i: (i, 0)),
            pl.BlockSpec((D, FF), lambda i: (0, 0)),
            pl.BlockSpec((1, FF), lambda i: (0, 0)),
            pl.BlockSpec((FF, 1), lambda i: (0, 0)),
            pl.BlockSpec((1, 1), lambda i: (0, 0)),
        ],
        out_specs=pl.BlockSpec((BM, 1), lambda i: (i, 0)),
        out_shape=jax.ShapeDtypeStruct((B, 1), jnp.float32),
    )(x, W1, b1.reshape(1, FF), W2, b2.reshape(1, 1))


def kernel(tokens, embed, W1, b1, W2, b2):
    embed_t = embed.T
    aux = jnp.pad(embed_t[:, TAIL0:], ((0, 0), (0, 4 * VB4 - (V - TAIL0))))
    table_lin = _fmt_tc(embed_t, aux).reshape(V4, D)
    pooled = _pool_sc(tokens.astype(jnp.int32), table_lin).reshape(B, D)
    out = _mlp_tc(pooled, W1, b1, W2, b2)
    return out[:, 0]
